# Initial kernel scaffold; baseline (speedup 1.0000x reference)
import functools

import jax
import jax.numpy as jnp
from jax import lax
from jax.experimental import pallas as pl
from jax.experimental.pallas import tpu as pltpu
from jax.experimental.pallas import tpu_sc as plsc

VOCAB = 1000
T = 95
D = 64
BATCH = 16384
N = BATCH * T

NC, NS, L = 2, 16, 16
NW = NC * NS
NT = N // NW
CHUNK = 512
NCHUNKS = NT // CHUNK
GATHER_ROWS = 128
NGATHER = CHUNK // GATHER_ROWS

_VB = 100


def _combine_body(tok_ref, pos_ref, out_ref):
    out_ref[...] = tok_ref[...][:, None, :] + pos_ref[...][None, :, :]


def _build_combined(token_embedding, position_embedding):
    return pl.pallas_call(
        _combine_body,
        grid=(VOCAB // _VB,),
        in_specs=[
            pl.BlockSpec((_VB, D), lambda i: (i, 0)),
            pl.BlockSpec((T, D), lambda i: (0, 0)),
        ],
        out_specs=pl.BlockSpec((_VB, T, D), lambda i: (i, 0, 0)),
        out_shape=jax.ShapeDtypeStruct((VOCAB, T, D), jnp.float32),
    )(token_embedding, position_embedding)


def _sc_gather(ids_flat, combined2d):
    mesh = plsc.VectorSubcoreMesh(core_axis_name="c", subcore_axis_name="s")

    @functools.partial(
        pl.kernel,
        mesh=mesh,
        out_type=jax.ShapeDtypeStruct((N, D), jnp.float32),
        scratch_types=[
            pltpu.VMEM((CHUNK,), jnp.int32),
            pltpu.VMEM((NGATHER, GATHER_ROWS), jnp.int32),
            pltpu.VMEM((CHUNK, D), jnp.float32),
            pltpu.SemaphoreType.DMA,
        ],
    )
    def k(ids_hbm, comb_hbm, out_hbm, ids_v, fid_v, rows_v, sem):
        wid = lax.axis_index("s") * NC + lax.axis_index("c")
        wbase = wid * NT
        iota = lax.broadcasted_iota(jnp.int32, (L,), 0)

        def body(c, carry):
            row0 = wbase + c * CHUNK
            pltpu.sync_copy(ids_hbm.at[pl.ds(row0, CHUNK)], ids_v)
            for g in range(CHUNK // L):
                idv = ids_v[pl.ds(g * L, L)]
                r = iota + (row0 + g * L)
                t = lax.rem(r, T)
                fid = idv * T + t
                fid_v[g * L // GATHER_ROWS,
                      pl.ds((g * L) % GATHER_ROWS, L)] = fid
            copies = [
                pltpu.async_copy(
                    comb_hbm.at[fid_v.at[j]],
                    rows_v.at[pl.ds(j * GATHER_ROWS, GATHER_ROWS)],
                    sem,
                )
                for j in range(NGATHER)
            ]
            for cp in copies:
                cp.wait()
            pltpu.sync_copy(rows_v, out_hbm.at[pl.ds(row0, CHUNK)])
            return carry

        lax.fori_loop(0, NCHUNKS, body, 0)

    return k(ids_flat, combined2d)


def kernel(position_token_ids, token_embedding, position_embedding):
    combined = _build_combined(token_embedding, position_embedding)
    comb2d = combined.reshape(VOCAB * T, D)
    ids_flat = position_token_ids.reshape(N).astype(jnp.int32)
    out_flat = _sc_gather(ids_flat, comb2d)
    return out_flat.reshape(BATCH, T, D)


# speedup vs baseline: 10.9410x; 10.9410x over previous
import functools

import jax
import jax.numpy as jnp
from jax import lax
from jax.experimental import pallas as pl
from jax.experimental.pallas import tpu as pltpu
from jax.experimental.pallas import tpu_sc as plsc

VOCAB = 1000
T = 95
D = 64
BATCH = 16384
N = BATCH * T

NC, NS, L = 2, 16, 16
NW = NC * NS
NT = N // NW
CHUNK = 512
NCHUNKS = NT // CHUNK
GATHER_ROWS = 128
NGATHER = CHUNK // GATHER_ROWS

_VB = 200


def _combine_body(tok_ref, pos_ref, out_ref):
    out_ref[...] = tok_ref[...][:, None, :] + pos_ref[...][None, :, :]


def _build_combined(token_embedding, position_embedding):
    return pl.pallas_call(
        _combine_body,
        grid=(VOCAB // _VB,),
        in_specs=[
            pl.BlockSpec((_VB, D), lambda i: (i, 0)),
            pl.BlockSpec((T, D), lambda i: (0, 0)),
        ],
        out_specs=pl.BlockSpec((_VB, T, D), lambda i: (i, 0, 0)),
        out_shape=jax.ShapeDtypeStruct((VOCAB, T, D), jnp.float32),
    )(token_embedding, position_embedding)


def _sc_gather(ids_flat, combined2d):
    mesh = plsc.VectorSubcoreMesh(core_axis_name="c", subcore_axis_name="s")

    @functools.partial(
        pl.kernel,
        mesh=mesh,
        compiler_params=pltpu.CompilerParams(use_tc_tiling_on_sc=False),
        out_type=jax.ShapeDtypeStruct((N, D), jnp.float32),
        scratch_types=[
            pltpu.VMEM((CHUNK,), jnp.int32),
            pltpu.VMEM((NGATHER, GATHER_ROWS), jnp.int32),
            pltpu.VMEM((CHUNK, D), jnp.float32),
            pltpu.SemaphoreType.DMA,
        ],
    )
    def k(ids_hbm, comb_hbm, out_hbm, ids_v, fid_v, rows_v, sem):
        wid = lax.axis_index("s") * NC + lax.axis_index("c")
        wbase = wid * NT
        iota = lax.broadcasted_iota(jnp.int32, (L,), 0)

        def body(c, carry):
            row0 = wbase + c * CHUNK
            pltpu.sync_copy(ids_hbm.at[pl.ds(row0, CHUNK)], ids_v)
            for g in range(CHUNK // L):
                idv = ids_v[pl.ds(g * L, L)]
                r = iota + (row0 + g * L)
                t = lax.rem(r, T)
                fid = idv * T + t
                fid_v[g * L // GATHER_ROWS,
                      pl.ds((g * L) % GATHER_ROWS, L)] = fid
            copies = [
                pltpu.async_copy(
                    comb_hbm.at[fid_v.at[j]],
                    rows_v.at[pl.ds(j * GATHER_ROWS, GATHER_ROWS)],
                    sem,
                )
                for j in range(NGATHER)
            ]
            for cp in copies:
                cp.wait()
            pltpu.sync_copy(rows_v, out_hbm.at[pl.ds(row0, CHUNK)])
            return carry

        lax.fori_loop(0, NCHUNKS, body, 0)

    return k(ids_flat, combined2d)


def kernel(position_token_ids, token_embedding, position_embedding):
    combined = _build_combined(token_embedding, position_embedding)
    comb2d = combined.reshape(VOCAB * T, D)
    ids_flat = position_token_ids.reshape(N).astype(jnp.int32)
    out_flat = _sc_gather(ids_flat, comb2d)
    return out_flat.reshape(BATCH, T, D)


# speedup vs baseline: 11.9551x; 1.0927x over previous
import functools

import jax
import jax.numpy as jnp
from jax import lax
from jax.experimental import pallas as pl
from jax.experimental.pallas import tpu as pltpu
from jax.experimental.pallas import tpu_sc as plsc

VOCAB = 1000
T = 95
D = 64
BATCH = 16384
N = BATCH * T

NC, NS, L = 2, 16, 16
NW = NC * NS
NT = N // NW
CHUNK = 320
NCHUNKS = NT // CHUNK
GR = 64
NG = CHUNK // GR
NBUF = 4

_VB = 200


def _combine_body(tok_ref, pos_ref, out_ref):
    out_ref[...] = tok_ref[...][:, None, :] + pos_ref[...][None, :, :]


def _build_combined(token_embedding, position_embedding):
    return pl.pallas_call(
        _combine_body,
        grid=(VOCAB // _VB,),
        in_specs=[
            pl.BlockSpec((_VB, D), lambda i: (i, 0)),
            pl.BlockSpec((T, D), lambda i: (0, 0)),
        ],
        out_specs=pl.BlockSpec((_VB, T, D), lambda i: (i, 0, 0)),
        out_shape=jax.ShapeDtypeStruct((VOCAB, T, D), jnp.float32),
    )(token_embedding, position_embedding)


def _sc_gather(ids_flat, combined2d):
    mesh = plsc.VectorSubcoreMesh(core_axis_name="c", subcore_axis_name="s")

    scratch = (
        [pltpu.VMEM((CHUNK,), jnp.int32) for _ in range(NBUF)]
        + [pltpu.VMEM((NG, GR), jnp.int32) for _ in range(NBUF)]
        + [pltpu.VMEM((CHUNK, D), jnp.float32) for _ in range(NBUF)]
        + [pltpu.SemaphoreType.DMA for _ in range(2 * NBUF)]
    )

    @functools.partial(
        pl.kernel,
        mesh=mesh,
        compiler_params=pltpu.CompilerParams(use_tc_tiling_on_sc=False),
        out_type=jax.ShapeDtypeStruct((N, D), jnp.float32),
        scratch_types=scratch,
    )
    def k(ids_hbm, comb_hbm, out_hbm, *sc):
        ids_v = sc[0:NBUF]
        fid_v = sc[NBUF:2 * NBUF]
        rows_v = sc[2 * NBUF:3 * NBUF]
        gsem = sc[3 * NBUF:3 * NBUF + NBUF]
        osem = sc[3 * NBUF + NBUF:]

        wid = lax.axis_index("s") * NC + lax.axis_index("c")
        wbase = wid * NT
        iota = lax.broadcasted_iota(jnp.int32, (L,), 0)

        def load_index(kk, b):
            r0 = wbase + kk * CHUNK
            pltpu.sync_copy(ids_hbm.at[pl.ds(r0, CHUNK)], ids_v[b])
            for g in range(CHUNK // L):
                idv = ids_v[b][pl.ds(g * L, L)]
                r = iota + (r0 + g * L)
                fid_v[b][g * L // GR, pl.ds((g * L) % GR, L)] = (
                    idv * T + lax.rem(r, T))

        def g_copies(b):
            return [
                pltpu.make_async_copy(
                    comb_hbm.at[fid_v[b].at[j]],
                    rows_v[b].at[pl.ds(j * GR, GR)],
                    gsem[b],
                )
                for j in range(NG)
            ]

        def o_copy(kk, b):
            return pltpu.make_async_copy(
                rows_v[b], out_hbm.at[pl.ds(wbase + kk * CHUNK, CHUNK)],
                osem[b])

        def step(c, bp, bq):
            f = c + NBUF - 1
            load_index(f, bq)
            o_copy(f - NBUF, bq).wait()
            for cp in g_copies(bq):
                cp.start()
            for cp in g_copies(bp):
                cp.wait()
            o_copy(c, bp).start()

        for kk in range(NBUF):
            load_index(kk, kk)
            for cp in g_copies(kk):
                cp.start()
        for cp in g_copies(0):
            cp.wait()
        o_copy(0, 0).start()

        def body(i, carry):
            c0 = 1 + i * NBUF
            step(c0, 1, 0)
            step(c0 + 1, 2, 1)
            step(c0 + 2, 3, 2)
            step(c0 + 3, 0, 3)
            return carry

        lax.fori_loop(0, (NCHUNKS - NBUF) // NBUF, body, 0)

        for c in range(NCHUNKS - NBUF + 1, NCHUNKS):
            for cp in g_copies(c % NBUF):
                cp.wait()
            o_copy(c, c % NBUF).start()
        for c in range(NCHUNKS - NBUF, NCHUNKS):
            o_copy(c, c % NBUF).wait()

    return k(ids_flat, combined2d)


def kernel(position_token_ids, token_embedding, position_embedding):
    combined = _build_combined(token_embedding, position_embedding)
    comb2d = combined.reshape(VOCAB * T, D)
    ids_flat = position_token_ids.reshape(N).astype(jnp.int32)
    out_flat = _sc_gather(ids_flat, comb2d)
    return out_flat.reshape(BATCH, T, D)
